# baseline (speedup 1.0000x reference)
import jax
import jax.numpy as jnp
from jax import lax
from jax.experimental import pallas as pl
from jax.experimental.pallas import tpu as pltpu
from jax.experimental.pallas import tpu_sc as plsc

MAXV = 100000
D = 32
HW = 256
NIDX = 3 * HW
NC, NS = 2, 16
NW = NC * NS
B = 1024
IMGS_PER_W = B // NW


def _sc_body(in_hbm, table_hbm, out_hbm, idx_v, rows_v, out_v, gsems, osem):
    wid = lax.axis_index("s") * NC + lax.axis_index("c")
    iota = lax.iota(jnp.int32, 16)
    base = wid * IMGS_PER_W

    def stage(par, img):
        pltpu.sync_copy(in_hbm.at[pl.ds(img * NIDX, NIDX)], idx_v.at[par])
        for j in range(6):
            pltpu.async_copy(
                table_hbm.at[idx_v.at[par, pl.ds(j * 128, 128)]],
                rows_v.at[par, pl.ds(j * 128, 128)],
                gsems[par],
            )

    def wait_gathers(par):
        for j in range(6):
            pltpu.make_async_copy(
                table_hbm.at[idx_v.at[par, pl.ds(j * 128, 128)]],
                rows_v.at[par, pl.ds(j * 128, 128)],
                gsems[par],
            ).wait()

    stage(0, base)

    @pl.loop(0, IMGS_PER_W // 2)
    def _pair(kk):
        for par in range(2):
            k = kk * 2 + par
            img = base + k

            @pl.when(k + 1 < IMGS_PER_W)
            def _():
                stage(1 - par, img + 1)

            wait_gathers(par)

            @pl.when(k >= 2)
            def _():
                pltpu.make_async_copy(
                    out_v.at[par, :, pl.ds(0, 128)],
                    out_hbm.at[img - 2],
                    osem,
                ).wait()

            @pl.loop(0, HW)
            def _acc(p):
                s0 = pl.ds(0, 16)
                s1 = pl.ds(16, 16)
                a0 = (
                    rows_v[par, p, s0]
                    + rows_v[par, p + 256, s0]
                    + rows_v[par, p + 512, s0]
                )
                a1 = (
                    rows_v[par, p, s1]
                    + rows_v[par, p + 256, s1]
                    + rows_v[par, p + 512, s1]
                )
                row = 2 * iota + (p >> 7)
                col = jnp.full((16,), p & 127, jnp.int32)
                plsc.store_scatter(out_v.at[par], [row, col], a0)
                plsc.store_scatter(out_v.at[par], [row + 32, col], a1)

            pltpu.async_copy(
                out_v.at[par, :, pl.ds(0, 128)],
                out_hbm.at[img],
                osem,
            )

    for par in range(2):
        img = base + IMGS_PER_W - 2 + par
        pltpu.make_async_copy(
            out_v.at[par, :, pl.ds(0, 128)],
            out_hbm.at[img],
            osem,
        ).wait()


REPACK_BLK = 128


def _repack_body(in_ref, out_ref):
    x = in_ref[...]
    parts = []
    for c in range(3):
        for r2 in range(2):
            row = jnp.concatenate(
                [x[:, c, 8 * r2 + h, :] for h in range(8)], axis=-1
            )
            parts.append((row + jnp.int32(c * MAXV))[:, None, :])
    y = jnp.concatenate(parts, axis=1)
    out_ref[...] = y.reshape(REPACK_BLK * 6, 128)


def _repack(inputs):
    return pl.pallas_call(
        _repack_body,
        grid=(B // REPACK_BLK,),
        in_specs=[
            pl.BlockSpec((REPACK_BLK, 3, 16, 16), lambda i: (i, 0, 0, 0))
        ],
        out_specs=pl.BlockSpec((REPACK_BLK * 6, 128), lambda i: (i, 0)),
        out_shape=jax.ShapeDtypeStruct((B * 6, 128), jnp.int32),
    )(inputs)


@jax.jit
def _bow_embed(inputs, table):
    in1d = _repack(inputs).reshape(B * NIDX)
    f = pl.kernel(
        _sc_body,
        out_type=jax.ShapeDtypeStruct((B, 64, 128), jnp.float32),
        mesh=plsc.VectorSubcoreMesh(core_axis_name="c", subcore_axis_name="s"),
        compiler_params=pltpu.CompilerParams(
            needs_layout_passes=False, use_tc_tiling_on_sc=False
        ),
        scratch_types=[
            pltpu.VMEM((2, NIDX), jnp.int32),
            pltpu.VMEM((2, NIDX, D), jnp.float32),
            pltpu.VMEM((2, 64, 129), jnp.float32),
            [pltpu.SemaphoreType.DMA, pltpu.SemaphoreType.DMA],
            pltpu.SemaphoreType.DMA,
        ],
    )
    return f(in1d, table)


def kernel(inputs, table):
    out = _bow_embed(inputs, table)
    return out.reshape(B, D, 16, 16)


# speedup vs baseline: 1.0169x; 1.0169x over previous
import jax
import jax.numpy as jnp
from jax import lax
from jax.experimental import pallas as pl
from jax.experimental.pallas import tpu as pltpu
from jax.experimental.pallas import tpu_sc as plsc

MAXV = 100000
D = 32
HW = 256
NIDX = 3 * HW
NC, NS = 2, 16
NW = NC * NS
B = 1024
IMGS_PER_W = B // NW


def _sc_body(in_hbm, table_hbm, out_hbm, idx_v, rows_v, out_v, gsems, osem):
    wid = lax.axis_index("s") * NC + lax.axis_index("c")
    iota = lax.iota(jnp.int32, 16)
    base = wid * IMGS_PER_W

    def stage(par, img):
        pltpu.sync_copy(in_hbm.at[pl.ds(img * NIDX, NIDX)], idx_v.at[par])
        for j in range(6):
            pltpu.async_copy(
                table_hbm.at[idx_v.at[par, pl.ds(j * 128, 128)]],
                rows_v.at[par, pl.ds(j * 128, 128)],
                gsems[par],
            )

    def wait_gathers(par):
        for j in range(6):
            pltpu.make_async_copy(
                table_hbm.at[idx_v.at[par, pl.ds(j * 128, 128)]],
                rows_v.at[par, pl.ds(j * 128, 128)],
                gsems[par],
            ).wait()

    stage(0, base)

    @pl.loop(0, IMGS_PER_W // 2)
    def _pair(kk):
        for par in range(2):
            k = kk * 2 + par
            img = base + k

            @pl.when(k + 1 < IMGS_PER_W)
            def _():
                stage(1 - par, img + 1)

            wait_gathers(par)

            @pl.when(k >= 2)
            def _():
                pltpu.make_async_copy(
                    out_v.at[par], out_hbm.at[img - 2], osem
                ).wait()

            @pl.loop(0, 512)
            def _acc(t):
                g = t >> 5
                d0 = t & 31
                pvec = g * 16 + iota
                dvec = (d0 + iota) & 31
                e = (
                    plsc.load_gather(rows_v.at[par], [pvec, dvec])
                    + plsc.load_gather(rows_v.at[par], [pvec + 256, dvec])
                    + plsc.load_gather(rows_v.at[par], [pvec + 512, dvec])
                )
                rowv = 2 * dvec + (g >> 3)
                colv = pvec & 127
                plsc.store_scatter(out_v.at[par], [rowv, colv], e)

            pltpu.async_copy(out_v.at[par], out_hbm.at[img], osem)

    for par in range(2):
        img = base + IMGS_PER_W - 2 + par
        pltpu.make_async_copy(
            out_v.at[par], out_hbm.at[img], osem
        ).wait()


REPACK_BLK = 128


def _repack_body(in_ref, out_ref):
    x = in_ref[...]
    parts = []
    for c in range(3):
        for r2 in range(2):
            row = jnp.concatenate(
                [x[:, c, 8 * r2 + h, :] for h in range(8)], axis=-1
            )
            parts.append((row + jnp.int32(c * MAXV))[:, None, :])
    y = jnp.concatenate(parts, axis=1)
    out_ref[...] = y.reshape(REPACK_BLK * 6, 128)


def _repack(inputs):
    return pl.pallas_call(
        _repack_body,
        grid=(B // REPACK_BLK,),
        in_specs=[
            pl.BlockSpec((REPACK_BLK, 3, 16, 16), lambda i: (i, 0, 0, 0))
        ],
        out_specs=pl.BlockSpec((REPACK_BLK * 6, 128), lambda i: (i, 0)),
        out_shape=jax.ShapeDtypeStruct((B * 6, 128), jnp.int32),
    )(inputs)


@jax.jit
def _bow_embed(inputs, table):
    in1d = _repack(inputs).reshape(B * NIDX)
    f = pl.kernel(
        _sc_body,
        out_type=jax.ShapeDtypeStruct((B, 64, 128), jnp.float32),
        mesh=plsc.VectorSubcoreMesh(core_axis_name="c", subcore_axis_name="s"),
        compiler_params=pltpu.CompilerParams(
            needs_layout_passes=False, use_tc_tiling_on_sc=False
        ),
        scratch_types=[
            pltpu.VMEM((2, NIDX), jnp.int32),
            pltpu.VMEM((2, NIDX, D), jnp.float32),
            pltpu.VMEM((2, 64, 128), jnp.float32),
            [pltpu.SemaphoreType.DMA, pltpu.SemaphoreType.DMA],
            pltpu.SemaphoreType.DMA,
        ],
    )
    return f(in1d, table)


def kernel(inputs, table):
    out = _bow_embed(inputs, table)
    return out.reshape(B, D, 16, 16)


# speedup vs baseline: 1.0915x; 1.0734x over previous
import jax
import jax.numpy as jnp
from jax import lax
from jax.experimental import pallas as pl
from jax.experimental.pallas import tpu as pltpu
from jax.experimental.pallas import tpu_sc as plsc

MAXV = 100000
D = 32
HW = 256
NIDX = 3 * HW
NC, NS = 2, 16
NW = NC * NS
B = 1024
IMGS_PER_W = B // NW


def _sc_body(in_hbm, table_hbm, out_hbm, idx_v, rows_v, out_v, gsems, osem):
    wid = lax.axis_index("s") * NC + lax.axis_index("c")
    iota = lax.iota(jnp.int32, 16)
    base = wid * IMGS_PER_W

    def stage(par, img):
        pltpu.sync_copy(in_hbm.at[pl.ds(img * NIDX, NIDX)], idx_v.at[par])
        for j in range(6):
            pltpu.async_copy(
                table_hbm.at[idx_v.at[par, pl.ds(j * 128, 128)]],
                rows_v.at[par, pl.ds(j * 128, 128)],
                gsems[par],
            )

    def wait_gathers(par):
        for j in range(6):
            pltpu.make_async_copy(
                table_hbm.at[idx_v.at[par, pl.ds(j * 128, 128)]],
                rows_v.at[par, pl.ds(j * 128, 128)],
                gsems[par],
            ).wait()

    stage(0, base)

    @pl.loop(0, IMGS_PER_W // 2)
    def _pair(kk):
        for par in range(2):
            k = kk * 2 + par
            img = base + k

            @pl.when(k + 1 < IMGS_PER_W)
            def _():
                stage(1 - par, img + 1)

            wait_gathers(par)

            @pl.when(k >= 2)
            def _():
                pltpu.make_async_copy(
                    out_v.at[par], out_hbm.at[img - 2], osem
                ).wait()

            @pl.loop(0, 512)
            def _acc(t):
                g = t >> 5
                d0 = t & 31
                pvec = g * 16 + iota
                dvec = (d0 + iota) & 31
                e = (
                    plsc.load_gather(rows_v.at[par], [pvec, dvec])
                    + plsc.load_gather(rows_v.at[par], [pvec + 256, dvec])
                    + plsc.load_gather(rows_v.at[par], [pvec + 512, dvec])
                )
                rowv = 2 * dvec + (g >> 3)
                colv = pvec & 127
                plsc.store_scatter(out_v.at[par], [rowv, colv], e)

            pltpu.async_copy(out_v.at[par], out_hbm.at[img], osem)

    for par in range(2):
        img = base + IMGS_PER_W - 2 + par
        pltpu.make_async_copy(
            out_v.at[par], out_hbm.at[img], osem
        ).wait()


REPACK_BLK = 128


def _repack_body(in_ref, out_ref):
    x = in_ref[...]
    y = x.reshape(NIDX, REPACK_BLK)
    z = jnp.transpose(y).reshape(REPACK_BLK * 6, 128)
    rr = jax.lax.broadcasted_iota(jnp.int32, (REPACK_BLK * 6, 128), 0)
    out_ref[...] = z + (rr % 6) // 2 * MAXV


def _repack(inputs_t):
    return pl.pallas_call(
        _repack_body,
        grid=(B // REPACK_BLK,),
        in_specs=[
            pl.BlockSpec((3, 16, 16, REPACK_BLK), lambda i: (0, 0, 0, i))
        ],
        out_specs=pl.BlockSpec((REPACK_BLK * 6, 128), lambda i: (i, 0)),
        out_shape=jax.ShapeDtypeStruct((B * 6, 128), jnp.int32),
    )(inputs_t)


@jax.jit
def _bow_embed(inputs, table):
    in1d = _repack(jnp.transpose(inputs, (1, 2, 3, 0))).reshape(B * NIDX)
    f = pl.kernel(
        _sc_body,
        out_type=jax.ShapeDtypeStruct((B, 64, 128), jnp.float32),
        mesh=plsc.VectorSubcoreMesh(core_axis_name="c", subcore_axis_name="s"),
        compiler_params=pltpu.CompilerParams(
            needs_layout_passes=False, use_tc_tiling_on_sc=False
        ),
        scratch_types=[
            pltpu.VMEM((2, NIDX), jnp.int32),
            pltpu.VMEM((2, NIDX, D), jnp.float32),
            pltpu.VMEM((2, 64, 128), jnp.float32),
            [pltpu.SemaphoreType.DMA, pltpu.SemaphoreType.DMA],
            pltpu.SemaphoreType.DMA,
        ],
    )
    return f(in1d, table)


def kernel(inputs, table):
    out = _bow_embed(inputs, table)
    return out.reshape(B, D, 16, 16)
